# baseline (speedup 1.0000x reference)
import functools

import jax
import jax.numpy as jnp
from jax import lax
from jax.experimental import pallas as pl
from jax.experimental.pallas import tpu as pltpu
from jax.experimental.pallas import tpu_sc as plsc

_NC = 2
_NS = 16
_NW = _NC * _NS


def _gather_sc(table, idx_t, V, D, J, B):
    W = B // _NW
    NTB = W // 128
    NF = D // 8
    mesh = plsc.VectorSubcoreMesh(core_axis_name="c", subcore_axis_name="s")

    @functools.partial(
        pl.kernel,
        mesh=mesh,
        out_type=jax.ShapeDtypeStruct((J, NF, B // 128, 8, 128), jnp.float32),
        scratch_types=[
            pltpu.VMEM((2, W), jnp.int32),
            pltpu.VMEM((2, W, D), jnp.float32),
            pltpu.VMEM((2, D, W + 1), jnp.float32),
            [pltpu.SemaphoreType.DMA] * 2,
            [pltpu.SemaphoreType.DMA] * 2,
        ],
        compiler_params=pltpu.CompilerParams(
            use_tc_tiling_on_sc=False, needs_layout_passes=False
        ),
    )
    def k(table_hbm, idx_hbm, out_hbm, idx_v, rows_v, trans_v, gsem, wsem):
        wid = lax.axis_index("s") * _NC + lax.axis_index("c")
        bstart = wid * W
        lane = lax.iota(jnp.int32, 16)

        pltpu.sync_copy(idx_hbm.at[0, pl.ds(bstart, W)], idx_v.at[0])
        pltpu.async_copy(table_hbm.at[idx_v.at[0]], rows_v.at[0], gsem[0])

        def slot(j, p):
            pltpu.make_async_copy(
                table_hbm.at[idx_v.at[p]], rows_v.at[p], gsem[p]
            ).wait()

            @pl.when(j + 1 < J)
            def _():
                pltpu.sync_copy(
                    idx_hbm.at[j + 1, pl.ds(bstart, W)], idx_v.at[1 - p]
                )
                pltpu.async_copy(
                    table_hbm.at[idx_v.at[1 - p]], rows_v.at[1 - p],
                    gsem[1 - p],
                )

            @pl.when(j >= 2)
            def _():
                for tf in range(NF):
                    for t in range(NTB):
                        pltpu.make_async_copy(
                            trans_v.at[p, pl.ds(tf * 8, 8), pl.ds(t * 128, 128)],
                            out_hbm.at[0, tf, t, :, :],
                            wsem[p],
                        ).wait()

            @plsc.parallel_loop(0, W, unroll=8)
            def tstep(b):
                for g in range(D // 16):
                    fidx = g * 16 + lane
                    vals = rows_v[p, b, pl.ds(g * 16, 16)]
                    bidx = jnp.full((16,), 0, jnp.int32) + b
                    plsc.store_scatter(trans_v.at[p], [fidx, bidx], vals)

            for tf in range(NF):
                for t in range(NTB):
                    pltpu.async_copy(
                        trans_v.at[p, pl.ds(tf * 8, 8), pl.ds(t * 128, 128)],
                        out_hbm.at[j, tf, wid * NTB + t, :, :],
                        wsem[p],
                    )

        def body(jj, carry):
            for p in range(2):
                slot(jj * 2 + p, p)
            return carry

        lax.fori_loop(0, J // 2, body, 0)

        for p in range(2):
            for tf in range(NF):
                for t in range(NTB):
                    pltpu.make_async_copy(
                        trans_v.at[p, pl.ds(tf * 8, 8), pl.ds(t * 128, 128)],
                        out_hbm.at[0, tf, t, :, :],
                        wsem[p],
                    ).wait()

    return k(table, idx_t)


def kernel(table, indices):
    B, J = indices.shape
    V, D = table.shape
    idx_t = jnp.transpose(indices).astype(jnp.int32)
    out5 = _gather_sc(table, idx_t, V, D, J, B)
    return out5.transpose(2, 4, 0, 1, 3).reshape(B, J, D)


# speedup vs baseline: 1.4444x; 1.4444x over previous
import functools

import jax
import jax.numpy as jnp
from jax import lax
from jax.experimental import pallas as pl
from jax.experimental.pallas import tpu as pltpu
from jax.experimental.pallas import tpu_sc as plsc

_NC = 2
_NS = 16
_NW = _NC * _NS


_RS = 40


def _detile_sc(table_t, V, D):
    NF = D // 8
    NT = (V + 127) // 128
    BLK = 128 * _RS
    ITS = (NT + _NW - 1) // _NW
    ITS = ITS + (ITS % 2)
    mesh = plsc.VectorSubcoreMesh(core_axis_name="c", subcore_axis_name="s")

    @functools.partial(
        pl.kernel,
        mesh=mesh,
        out_type=jax.ShapeDtypeStruct((NT * BLK,), jnp.float32),
        scratch_types=[
            pltpu.VMEM((2, NF, 8, 128), jnp.float32),
            pltpu.VMEM((BLK,), jnp.float32),
            pltpu.VMEM((BLK,), jnp.float32),
            [pltpu.SemaphoreType.DMA] * 2,
            [pltpu.SemaphoreType.DMA] * 2,
        ],
        compiler_params=pltpu.CompilerParams(
            use_tc_tiling_on_sc=True, needs_layout_passes=False,
            disable_bounds_checks=True,
        ),
    )
    def k(tab_hbm, out_hbm, in_v, dst_v0, dst_v1, isem, osem):
        wid = lax.axis_index("s") * _NC + lax.axis_index("c")
        lane = lax.iota(jnp.int32, 16)
        base = [(k16 * 16 + lane) * _RS for k16 in range(8)]
        dst_v = (dst_v0, dst_v1)

        def tile_of(it):
            return jnp.minimum(it * _NW + wid, NT - 1)

        def fetch(it, p):
            t = tile_of(it)
            for tf in range(NF):
                pltpu.async_copy(
                    tab_hbm.at[pl.ds(tf * 8, 8), pl.ds(t * 128, 128)],
                    in_v.at[p, tf], isem[p],
                )

        fetch(0, 0)

        def step(it, p):
            for tf in range(NF):
                pltpu.make_async_copy(
                    tab_hbm.at[pl.ds(0, 8), pl.ds(0, 128)], in_v.at[p, tf],
                    isem[p],
                ).wait()

            @pl.when(it + 1 < ITS)
            def _():
                fetch(it + 1, 1 - p)

            @pl.when(it >= 2)
            def _():
                pltpu.make_async_copy(
                    dst_v[p], out_hbm.at[pl.ds(0, BLK)], osem[p]
                ).wait()

            for tf in range(NF):
                for f in range(8):
                    fg = tf * 8 + f
                    for k16 in range(8):
                        vals = in_v[p, tf, f, pl.ds(k16 * 16, 16)]
                        plsc.store_scatter(dst_v[p], [base[k16] + fg], vals)

            pltpu.async_copy(
                dst_v[p], out_hbm.at[pl.ds(tile_of(it) * BLK, BLK)], osem[p]
            )

        def body(jj, carry):
            for p in range(2):
                step(jj * 2 + p, p)
            return carry

        lax.fori_loop(0, ITS // 2, body, 0)
        for p in range(2):
            pltpu.make_async_copy(
                dst_v[p], out_hbm.at[pl.ds(0, BLK)], osem[p]
            ).wait()

    return k(table_t)


def _gather_sc(table, idx_t, RW, D, J, B):
    W = B // _NW
    NTB = W // 128
    NF = D // 8
    mesh = plsc.VectorSubcoreMesh(core_axis_name="c", subcore_axis_name="s")

    @functools.partial(
        pl.kernel,
        mesh=mesh,
        out_type=jax.ShapeDtypeStruct((J, NF, B // 128, 8, 128), jnp.float32),
        scratch_types=[
            pltpu.VMEM((2, W), jnp.int32),
            pltpu.VMEM((2, W, RW), jnp.float32),
            pltpu.VMEM((2, D, W + 1), jnp.float32),
            [pltpu.SemaphoreType.DMA] * 2,
            [pltpu.SemaphoreType.DMA] * 2,
        ],
        compiler_params=pltpu.CompilerParams(
            use_tc_tiling_on_sc=False, needs_layout_passes=False
        ),
    )
    def k(table_hbm, idx_hbm, out_hbm, idx_v, rows_v, trans_v, gsem, wsem):
        wid = lax.axis_index("s") * _NC + lax.axis_index("c")
        bstart = wid * W
        lane = lax.iota(jnp.int32, 16)

        pltpu.sync_copy(idx_hbm.at[0, pl.ds(bstart, W)], idx_v.at[0])
        pltpu.async_copy(table_hbm.at[idx_v.at[0]], rows_v.at[0], gsem[0])

        def slot(j, p):
            pltpu.make_async_copy(
                table_hbm.at[idx_v.at[p]], rows_v.at[p], gsem[p]
            ).wait()

            @pl.when(j + 1 < J)
            def _():
                pltpu.sync_copy(
                    idx_hbm.at[j + 1, pl.ds(bstart, W)], idx_v.at[1 - p]
                )
                pltpu.async_copy(
                    table_hbm.at[idx_v.at[1 - p]], rows_v.at[1 - p],
                    gsem[1 - p],
                )

            @pl.when(j >= 2)
            def _():
                for tf in range(NF):
                    for t in range(NTB):
                        pltpu.make_async_copy(
                            trans_v.at[p, pl.ds(tf * 8, 8), pl.ds(t * 128, 128)],
                            out_hbm.at[0, tf, t, :, :],
                            wsem[p],
                        ).wait()

            @plsc.parallel_loop(0, W, unroll=8)
            def tstep(b):
                for g in range(D // 16):
                    fidx = g * 16 + lane
                    vals = rows_v[p, b, pl.ds(g * 16, 16)]
                    bidx = jnp.full((16,), 0, jnp.int32) + b
                    plsc.store_scatter(trans_v.at[p], [fidx, bidx], vals)

            for tf in range(NF):
                for t in range(NTB):
                    pltpu.async_copy(
                        trans_v.at[p, pl.ds(tf * 8, 8), pl.ds(t * 128, 128)],
                        out_hbm.at[j, tf, wid * NTB + t, :, :],
                        wsem[p],
                    )

        def body(jj, carry):
            for p in range(2):
                slot(jj * 2 + p, p)
            return carry

        lax.fori_loop(0, J // 2, body, 0)

        for p in range(2):
            for tf in range(NF):
                for t in range(NTB):
                    pltpu.make_async_copy(
                        trans_v.at[p, pl.ds(tf * 8, 8), pl.ds(t * 128, 128)],
                        out_hbm.at[0, tf, t, :, :],
                        wsem[p],
                    ).wait()

    return k(table, idx_t)


def kernel(table, indices):
    B, J = indices.shape
    V, D = table.shape
    idx_t = jnp.transpose(indices).astype(jnp.int32)
    table_t = jnp.transpose(table)
    flat33 = _detile_sc(table_t, V, D)
    table33 = flat33.reshape(flat33.shape[0] // _RS, _RS)
    out5 = _gather_sc(table33, idx_t, _RS, D, J, B)
    return out5.transpose(2, 4, 0, 1, 3).reshape(B, J, D)
